# Initial kernel scaffold; baseline (speedup 1.0000x reference)
import jax
import jax.numpy as jnp
from jax.experimental import pallas as pl


def kernel(x, tables):
    raise NotImplementedError("write your pallas kernel here")


# speedup vs baseline: 6.9535x; 6.9535x over previous
import jax
import jax.numpy as jnp
from jax import lax
from jax.experimental import pallas as pl
from jax.experimental.pallas import tpu as pltpu, tpu_sc as plsc

NC, NS, LANES = 2, 16, 16
NW = NC * NS

C = 128


def _sc_body(num_fields, vocab, dim, nrounds, tab_hbm, idx_hbm, out_hbm,
             idxb, rows, acc, sem):
    cid = lax.axis_index("c")
    sid = lax.axis_index("s")
    wid = sid * NC + cid
    tw = nrounds * C
    base = wid * tw

    def round_fn(r, _):
        tbase = base + r * C
        pltpu.sync_copy(idx_hbm.at[wid, r], idxb)

        pltpu.async_copy(tab_hbm.at[idxb.at[0]], acc, sem).wait()

        def field_fn(i, _):
            def off_fn(j, _):
                sl = pl.ds(j * LANES, LANES)
                idxb[i, sl] = idxb[i, sl] + i * vocab
                return 0
            lax.fori_loop(0, C // LANES, off_fn, 0, unroll=True)

            pltpu.async_copy(tab_hbm.at[idxb.at[i]], rows, sem).wait()

            def add_fn(t, _):
                lo = pl.ds(0, LANES)
                hi = pl.ds(LANES, LANES)
                acc[t, lo] = acc[t, lo] + rows[t, lo]
                acc[t, hi] = acc[t, hi] + rows[t, hi]
                return 0
            lax.fori_loop(0, C, add_fn, 0)
            return 0

        lax.fori_loop(1, num_fields, field_fn, 0)
        pltpu.sync_copy(acc, out_hbm.at[pl.ds(tbase, C)])
        return 0

    lax.fori_loop(0, nrounds, round_fn, 0)


def kernel(x, tables):
    B, L, F = x.shape
    _, V, D = tables.shape
    T = B * L
    assert T % (NW * C) == 0
    nrounds = T // (NW * C)

    idx_t = x.reshape(NW, nrounds, C, F).transpose(0, 1, 3, 2)
    tab_flat = tables.reshape(F * V, D)

    import functools
    run = pl.kernel(
        functools.partial(_sc_body, F, V, D, nrounds),
        out_type=jax.ShapeDtypeStruct((T, D), jnp.float32),
        mesh=plsc.VectorSubcoreMesh(core_axis_name="c", subcore_axis_name="s",
                                    num_cores=NC, num_subcores=NS),
        scratch_types=[
            pltpu.VMEM((F, C), jnp.int32),
            pltpu.VMEM((C, D), jnp.float32),
            pltpu.VMEM((C, D), jnp.float32),
            pltpu.SemaphoreType.DMA,
        ],
        compiler_params=pltpu.CompilerParams(use_tc_tiling_on_sc=False),
    )
    out = run(tab_flat, idx_t)
    return out.reshape(B, L, D)


# speedup vs baseline: 7.3672x; 1.0595x over previous
import functools
import jax
import jax.numpy as jnp
from jax import lax
from jax.experimental import pallas as pl
from jax.experimental.pallas import tpu as pltpu, tpu_sc as plsc

NC, NS, LANES = 2, 16, 16
NW = NC * NS

C = 128
NBUF = 4


def _sc_body(num_fields, vocab, dim, nrounds, tab_hbm, idx_hbm, out_hbm,
             idxb, rows, acc2, gsem, outsem):
    cid = lax.axis_index("c")
    sid = lax.axis_index("s")
    wid = sid * NC + cid
    tw = nrounds * C
    base = wid * tw

    def out_slot(r):
        return acc2.at[r % 2]

    def round_fn(r, _):
        tbase = base + r * C
        p = r % 2

        @pl.when(r >= 2)
        def _():
            pltpu.make_async_copy(
                acc2.at[p], out_hbm.at[pl.ds(tbase, C)], outsem.at[p]).wait()

        pltpu.sync_copy(idx_hbm.at[wid, r], idxb)

        def off_fn(i, _):
            def off_j(j, _):
                sl = pl.ds(j * LANES, LANES)
                idxb[i, sl] = idxb[i, sl] + i * vocab
                return 0
            lax.fori_loop(0, C // LANES, off_j, 0, unroll=True)
            return 0
        lax.fori_loop(1, num_fields, off_fn, 0)

        d0 = pltpu.async_copy(tab_hbm.at[idxb.at[0]], acc2.at[p],
                              gsem.at[NBUF])
        for b in range(NBUF):
            pltpu.async_copy(tab_hbm.at[idxb.at[1 + b]], rows.at[b],
                             gsem.at[b])
        d0.wait()

        def field_fn(i, _):
            b = (i - 1) % NBUF
            pltpu.make_async_copy(tab_hbm.at[idxb.at[i]], rows.at[b],
                                  gsem.at[b]).wait()

            def add_fn(t, _):
                lo = pl.ds(0, LANES)
                hi = pl.ds(LANES, LANES)
                acc2[p, t, lo] = acc2[p, t, lo] + rows[b, t, lo]
                acc2[p, t, hi] = acc2[p, t, hi] + rows[b, t, hi]
                return 0
            lax.fori_loop(0, C, add_fn, 0, unroll=4)

            @pl.when(i + NBUF < num_fields)
            def _():
                pltpu.async_copy(tab_hbm.at[idxb.at[i + NBUF]], rows.at[b],
                                 gsem.at[b])
            return 0
        lax.fori_loop(1, num_fields, field_fn, 0)

        pltpu.async_copy(acc2.at[p], out_hbm.at[pl.ds(tbase, C)],
                         outsem.at[p])
        return 0

    lax.fori_loop(0, nrounds, round_fn, 0)

    for p in range(2):
        pltpu.make_async_copy(
            acc2.at[p], out_hbm.at[pl.ds(base, C)], outsem.at[p]).wait()


def kernel(x, tables):
    B, L, F = x.shape
    _, V, D = tables.shape
    T = B * L
    assert T % (NW * C) == 0
    nrounds = T // (NW * C)
    assert nrounds >= 2 and nrounds % 2 == 0

    idx_t = x.reshape(NW, nrounds, C, F).transpose(0, 1, 3, 2)
    tab_flat = tables.reshape(F * V, D)

    run = pl.kernel(
        functools.partial(_sc_body, F, V, D, nrounds),
        out_type=jax.ShapeDtypeStruct((T, D), jnp.float32),
        mesh=plsc.VectorSubcoreMesh(core_axis_name="c", subcore_axis_name="s",
                                    num_cores=NC, num_subcores=NS),
        scratch_types=[
            pltpu.VMEM((F, C), jnp.int32),
            pltpu.VMEM((NBUF, C, D), jnp.float32),
            pltpu.VMEM((2, C, D), jnp.float32),
            pltpu.SemaphoreType.DMA((NBUF + 1,)),
            pltpu.SemaphoreType.DMA((2,)),
        ],
        compiler_params=pltpu.CompilerParams(use_tc_tiling_on_sc=False),
    )
    out = run(tab_flat, idx_t)
    return out.reshape(B, L, D)
